# baseline (speedup 1.0000x reference)
import jax
import jax.numpy as jnp
from jax import lax
from jax.experimental import pallas as pl
from jax.experimental.pallas import tpu as pltpu
from jax.experimental.pallas import tpu_sc as plsc

B, L, D = 4096, 200, 64
NV = 1_000_000
NC, NS = 2, 16
NW = NC * NS
LT = L // 8
CH = 256
NCHUNK = LT * 4
HALF = NCHUNK // 2
TIDX = D * CH // 16

VCH = 2048
_NSTEP = -(-NV // VCH)


def _fmt_body(wt_ref, out_ref):
    y = wt_ref[...].T.reshape(VCH // 2, 2, D)
    out_ref[...] = jnp.concatenate([y[:, 0, :], y[:, 1, :]], axis=1)


_format = pl.pallas_call(
    _fmt_body,
    out_shape=jax.ShapeDtypeStruct((NV // 2, 128), jnp.float32),
    grid=(_NSTEP,),
    in_specs=[pl.BlockSpec((D, VCH), lambda i: (0, i))],
    out_specs=pl.BlockSpec((VCH // 2, 128), lambda i: (i, 0)),
)

_mesh = plsc.VectorSubcoreMesh(
    core_axis_name="c", subcore_axis_name="s", num_cores=NC, num_subcores=NS
)


def _body(xt_hbm, w_hbm, out_hbm, xb0, xb1, rows0, rows1, t0, t1, tidx,
          sem_g0, sem_g1, sem_o0, sem_o1):
    wid = lax.axis_index("s") * NC + lax.axis_index("c")
    bcol = wid * 128

    iota = lax.iota(jnp.int32, 16)

    @plsc.parallel_loop(0, TIDX, unroll=4)
    def _init_table(kk):
        tidx[pl.ds(16 * kk, 16)] = ((kk % 4) * 16 + iota) * (CH + 1) + kk // 4

    def load_idx(c, xbuf):
        lrow = (c // 4) * 8 + (c % 4) * 2
        pltpu.sync_copy(xt_hbm.at[pl.ds(lrow, 2), pl.ds(bcol, 128)], xbuf)

    def start_gather(xbuf, rows, sem):
        pltpu.async_copy(w_hbm.at[xbuf.at[0]], rows.at[pl.ds(0, 128)], sem)
        pltpu.async_copy(w_hbm.at[xbuf.at[1]], rows.at[pl.ds(128, 128)], sem)

    def wait_gather(rows, sem):
        pltpu.make_async_copy(w_hbm.at[pl.ds(0, 128)],
                              rows.at[pl.ds(0, 128)], sem).wait()
        pltpu.make_async_copy(w_hbm.at[pl.ds(0, 128)],
                              rows.at[pl.ds(128, 128)], sem).wait()

    def start_out(tbuf, c, sem):
        lt = c // 4
        h = c % 4
        pltpu.async_copy(
            tbuf.at[:, pl.ds(0, CH)], out_hbm.at[:, lt, wid, pl.ds(h * CH, CH)], sem)

    def wait_out(tbuf, sem):
        pltpu.make_async_copy(out_hbm.at[:, 0, 0, pl.ds(0, CH)],
                              tbuf.at[:, pl.ds(0, CH)], sem).wait()

    zero16 = iota * 0

    def transpose_chunk(rows, tbuf):
        @plsc.parallel_loop(0, TIDX, unroll=8)
        def _(kk):
            v = rows[kk // 4, pl.ds((kk % 4) * 16, 16)]
            ix = tidx[pl.ds(16 * kk, 16)]
            plsc.store_scatter(tbuf, [zero16, ix], v)

    load_idx(0, xb0)
    start_gather(xb0, rows0, sem_g0)

    def body(i, _):
        cA = 2 * i
        cB = cA + 1
        load_idx(cB, xb1)
        start_gather(xb1, rows1, sem_g1)

        wait_gather(rows0, sem_g0)

        @pl.when(i > 0)
        def _():
            wait_out(t0, sem_o0)

        transpose_chunk(rows0, t0)
        start_out(t0, cA, sem_o0)

        @pl.when(i < HALF - 1)
        def _():
            load_idx(cA + 2, xb0)
            start_gather(xb0, rows0, sem_g0)

        wait_gather(rows1, sem_g1)

        @pl.when(i > 0)
        def _():
            wait_out(t1, sem_o1)

        transpose_chunk(rows1, t1)
        start_out(t1, cB, sem_o1)
        return 0

    lax.fori_loop(0, HALF, body, 0)
    wait_out(t0, sem_o0)
    wait_out(t1, sem_o1)


_fused = pl.kernel(
    _body,
    out_type=jax.ShapeDtypeStruct((D, LT, NW, 1024), jnp.float32),
    mesh=_mesh,
    scratch_types=[
        pltpu.VMEM((2, 128), jnp.int32),
        pltpu.VMEM((2, 128), jnp.int32),
        pltpu.VMEM((CH, D), jnp.float32),
        pltpu.VMEM((CH, D), jnp.float32),
        pltpu.VMEM((D, CH + 1), jnp.float32),
        pltpu.VMEM((D, CH + 1), jnp.float32),
        pltpu.VMEM((16 * TIDX,), jnp.int32),
        pltpu.SemaphoreType.DMA,
        pltpu.SemaphoreType.DMA,
        pltpu.SemaphoreType.DMA,
        pltpu.SemaphoreType.DMA,
    ],
    compiler_params=pltpu.CompilerParams(
        use_tc_tiling_on_sc=False, needs_layout_passes=False
    ),
)


def kernel(x, weight):
    xt = jnp.transpose(x, (1, 0)).astype(jnp.int32)
    wlin = _format(jnp.transpose(weight, (1, 0)))
    out = _fused(xt, wlin.reshape(NV, D))
    out = out.reshape(D, LT, NW, 8, 128)
    out = out.transpose(2, 4, 0, 1, 3)
    return out.reshape(B, D, L)


# speedup vs baseline: 1.2205x; 1.2205x over previous
import jax
import jax.numpy as jnp
from jax import lax
from jax.experimental import pallas as pl
from jax.experimental.pallas import tpu as pltpu
from jax.experimental.pallas import tpu_sc as plsc

B, L, D = 4096, 200, 64
NV = 1_000_000
NC, NS = 2, 16
NW = NC * NS
LT = L // 8
CH = 256
NCHUNK = LT * 4
HALF = NCHUNK // 2
TIDX = D * CH // 16

VCH = 16384
_NSTEP = -(-NV // VCH)


def _fmt_body(wt_ref, out_ref):
    y = wt_ref[...].T.reshape(VCH // 2, 2, D)
    out_ref[...] = jnp.concatenate([y[:, 0, :], y[:, 1, :]], axis=1)


_format = pl.pallas_call(
    _fmt_body,
    out_shape=jax.ShapeDtypeStruct((NV // 2, 128), jnp.float32),
    grid=(_NSTEP,),
    in_specs=[pl.BlockSpec((D, VCH), lambda i: (0, i))],
    out_specs=pl.BlockSpec((VCH // 2, 128), lambda i: (i, 0)),
)

_mesh = plsc.VectorSubcoreMesh(
    core_axis_name="c", subcore_axis_name="s", num_cores=NC, num_subcores=NS
)


def _body(xt_hbm, w_hbm, out_hbm, xb0, xb1, rows0, rows1, t0, t1, tidx,
          sem_g0, sem_g1, sem_o0, sem_o1):
    wid = lax.axis_index("s") * NC + lax.axis_index("c")
    bcol = wid * 128

    iota = lax.iota(jnp.int32, 16)

    @plsc.parallel_loop(0, TIDX, unroll=4)
    def _init_table(kk):
        tidx[pl.ds(16 * kk, 16)] = ((kk % 4) * 16 + iota) * (CH + 1) + kk // 4

    def load_idx(c, xbuf):
        lrow = (c // 4) * 8 + (c % 4) * 2
        pltpu.sync_copy(xt_hbm.at[pl.ds(lrow, 2), pl.ds(bcol, 128)], xbuf)

    def start_gather(xbuf, rows, sem):
        pltpu.async_copy(w_hbm.at[xbuf.at[0]], rows.at[pl.ds(0, 128)], sem)
        pltpu.async_copy(w_hbm.at[xbuf.at[1]], rows.at[pl.ds(128, 128)], sem)

    def wait_gather(rows, sem):
        pltpu.make_async_copy(w_hbm.at[pl.ds(0, 128)],
                              rows.at[pl.ds(0, 128)], sem).wait()
        pltpu.make_async_copy(w_hbm.at[pl.ds(0, 128)],
                              rows.at[pl.ds(128, 128)], sem).wait()

    def start_out(tbuf, c, sem):
        lt = c // 4
        h = c % 4
        pltpu.async_copy(
            tbuf.at[:, pl.ds(0, CH)], out_hbm.at[:, lt, wid, pl.ds(h * CH, CH)], sem)

    def wait_out(tbuf, sem):
        pltpu.make_async_copy(out_hbm.at[:, 0, 0, pl.ds(0, CH)],
                              tbuf.at[:, pl.ds(0, CH)], sem).wait()

    zero16 = iota * 0

    def transpose_chunk(rows, tbuf):
        @plsc.parallel_loop(0, TIDX, unroll=8)
        def _(kk):
            v = rows[kk // 4, pl.ds((kk % 4) * 16, 16)]
            ix = tidx[pl.ds(16 * kk, 16)]
            plsc.store_scatter(tbuf, [zero16, ix], v)

    load_idx(0, xb0)
    start_gather(xb0, rows0, sem_g0)

    def body(i, _):
        cA = 2 * i
        cB = cA + 1
        load_idx(cB, xb1)
        start_gather(xb1, rows1, sem_g1)

        wait_gather(rows0, sem_g0)

        @pl.when(i > 0)
        def _():
            wait_out(t0, sem_o0)

        transpose_chunk(rows0, t0)
        start_out(t0, cA, sem_o0)

        @pl.when(i < HALF - 1)
        def _():
            load_idx(cA + 2, xb0)
            start_gather(xb0, rows0, sem_g0)

        wait_gather(rows1, sem_g1)

        @pl.when(i > 0)
        def _():
            wait_out(t1, sem_o1)

        transpose_chunk(rows1, t1)
        start_out(t1, cB, sem_o1)
        return 0

    lax.fori_loop(0, HALF, body, 0)
    wait_out(t0, sem_o0)
    wait_out(t1, sem_o1)


_fused = pl.kernel(
    _body,
    out_type=jax.ShapeDtypeStruct((D, LT, NW, 1024), jnp.float32),
    mesh=_mesh,
    scratch_types=[
        pltpu.VMEM((2, 128), jnp.int32),
        pltpu.VMEM((2, 128), jnp.int32),
        pltpu.VMEM((CH, D), jnp.float32),
        pltpu.VMEM((CH, D), jnp.float32),
        pltpu.VMEM((D, CH + 1), jnp.float32),
        pltpu.VMEM((D, CH + 1), jnp.float32),
        pltpu.VMEM((16 * TIDX,), jnp.int32),
        pltpu.SemaphoreType.DMA,
        pltpu.SemaphoreType.DMA,
        pltpu.SemaphoreType.DMA,
        pltpu.SemaphoreType.DMA,
    ],
    compiler_params=pltpu.CompilerParams(
        use_tc_tiling_on_sc=False, needs_layout_passes=False
    ),
)


def kernel(x, weight):
    xt = jnp.transpose(x, (1, 0)).astype(jnp.int32)
    wlin = _format(jnp.transpose(weight, (1, 0)))
    out = _fused(xt, wlin.reshape(NV, D))
    out = out.reshape(D, LT, NW, 8, 128)
    out = out.transpose(2, 4, 0, 1, 3)
    return out.reshape(B, D, L)
